# Initial kernel scaffold; baseline (speedup 1.0000x reference)
import jax
import jax.numpy as jnp
from jax.experimental import pallas as pl


def kernel(inputs, label, lut, cq):
    raise NotImplementedError("write your pallas kernel here")


# speedup vs baseline: 11.7233x; 11.7233x over previous
import functools
import math

import jax
import jax.numpy as jnp
from jax.experimental import pallas as pl
from jax.experimental.pallas import tpu as pltpu
from jax.experimental.pallas import tpu_sc as plsc

N_FEAT = 128
N_LUT = 100000
N_CQ = 5000
SCAL = 30.0
B = 1024

BK = 1000
NB_LUT = N_LUT // BK
NB_CQ = N_CQ // BK
NSTEPS = NB_LUT + NB_CQ

EXP_P = math.exp(SCAL)
EXP_M = math.exp(-SCAL)

GATHER_WINDOW = 128


def _sc_gather(lut, idx2d):
    mesh = plsc.VectorSubcoreMesh(core_axis_name="c", subcore_axis_name="s")

    @functools.partial(
        pl.kernel,
        out_type=jax.ShapeDtypeStruct((B, N_FEAT), jnp.float32),
        mesh=mesh,
    )
    def kern(lut_hbm, i_hbm, o_hbm):
        def body(i_vmem, o_vmem):
            pltpu.sync_copy(lut_hbm.at[i_vmem.at[0]], o_vmem)

        pltpu.emit_pipeline(
            body,
            grid=(B // GATHER_WINDOW,),
            in_specs=[pl.BlockSpec((1, GATHER_WINDOW), index_map=lambda i: (0, i))],
            out_specs=[pl.BlockSpec((GATHER_WINDOW, N_FEAT),
                                    index_map=lambda i: (i, 0))],
            core_axis_name=("c", "s"),
            dimension_semantics=(pltpu.PARALLEL,),
        )(i_hbm, o_hbm)

    return kern(lut, idx2d)


def _tc_body(x_ref, lab_ref, lut_ref, cq_ref, g_ref, out_ref,
             xn_ref, xb_ref, sacc_ref, nbad_ref):
    i = pl.program_id(0)

    @pl.when(i == 0)
    def _init():
        x = x_ref[...]
        ss = jnp.sum(x * x, axis=1, keepdims=True)
        nrm = jnp.maximum(jnp.sqrt(ss), 1e-12)
        xn = x * (SCAL / nrm)
        xn_ref[...] = xn
        xb_ref[...] = xn.astype(jnp.bfloat16)
        sacc_ref[...] = jnp.zeros_like(sacc_ref)
        nbad_ref[0] = jnp.int32(0)

    ones8 = jnp.ones((8, N_FEAT), dtype=jnp.bfloat16)

    def process(w_ref):
        wb = w_ref[...].astype(jnp.bfloat16)
        z = jax.lax.dot_general(xb_ref[...], wb, (((1,), (1,)), ((), ())),
                                preferred_element_type=jnp.float32)
        sacc_ref[...] += jnp.sum(jnp.exp(z), axis=1, keepdims=True)
        colabs = jax.lax.dot_general(ones8, jnp.abs(wb),
                                     (((1,), (1,)), ((), ())),
                                     preferred_element_type=jnp.float32)
        nbad_ref[0] += jnp.sum((colabs[0:1, :] == 0.0).astype(jnp.int32))

    @pl.when(i < NB_LUT)
    def _lut():
        process(lut_ref)

    @pl.when(i >= NB_LUT)
    def _cq():
        process(cq_ref)

    @pl.when(i == NSTEPS - 1)
    def _final():
        nbad = nbad_ref[0].astype(jnp.float32)
        g = g_ref[...]
        t0 = jnp.sum(xn_ref[...] * g, axis=1, keepdims=True)
        gabs = jnp.sum(jnp.abs(g), axis=1, keepdims=True)
        bad_pos = gabs == 0.0
        s = sacc_ref[...] + nbad * (EXP_M - 1.0)
        s = s + jnp.where(bad_pos, EXP_P - EXP_M, 0.0)
        t = jnp.where(bad_pos, SCAL, t0)
        valid = lab_ref[...] != N_LUT
        li = jnp.where(valid, jnp.log(s) - t, 0.0)
        out_ref[...] = jnp.reshape(jnp.sum(li) * (1.0 / B), (1, 1))


def kernel(inputs, label, lut, cq):
    lab = label.astype(jnp.int32)
    safe = jnp.minimum(lab, N_LUT - 1).reshape(1, B)
    g = _sc_gather(lut, safe)

    out = pl.pallas_call(
        _tc_body,
        grid=(NSTEPS,),
        in_specs=[
            pl.BlockSpec((B, N_FEAT), lambda i: (0, 0)),
            pl.BlockSpec((B, 1), lambda i: (0, 0)),
            pl.BlockSpec((BK, N_FEAT),
                         lambda i: (jnp.minimum(i, NB_LUT - 1), 0)),
            pl.BlockSpec((BK, N_FEAT),
                         lambda i: (jnp.clip(i - NB_LUT, 0, NB_CQ - 1), 0)),
            pl.BlockSpec((B, N_FEAT), lambda i: (0, 0)),
        ],
        out_specs=pl.BlockSpec((1, 1), lambda i: (0, 0)),
        out_shape=jax.ShapeDtypeStruct((1, 1), jnp.float32),
        scratch_shapes=[
            pltpu.VMEM((B, N_FEAT), jnp.float32),
            pltpu.VMEM((B, N_FEAT), jnp.bfloat16),
            pltpu.VMEM((B, 1), jnp.float32),
            pltpu.SMEM((1,), jnp.int32),
        ],
    )(inputs, lab.reshape(B, 1), lut, cq, g)
    return out[0, 0]
